# baseline (speedup 1.0000x reference)
import functools

import jax
import jax.numpy as jnp
from jax import lax
from jax.experimental import pallas as pl
from jax.experimental.pallas import tpu as pltpu
from jax.experimental.pallas import tpu_sc as plsc

LATENT_DIM = 128
TIME_RES = 1000
TABLE_ROWS = 1024
IDX_CHUNK = 128
N_REP = 8


def _make_prep(n_chunks):
    def _prep_body(pe_ref, w1_ref, b1_ref, w2_ref, b2_ref, t_ref,
                   table_ref, idx_ref):
        x = pe_ref[:, 0, :]
        h = jnp.dot(x, w1_ref[...], preferred_element_type=jnp.float32)
        h = h + b1_ref[...]
        h = h * jax.nn.sigmoid(h)
        o = jnp.dot(h, w2_ref[...], preferred_element_type=jnp.float32)
        res = o + b2_ref[...]
        for k in range(N_REP):
            table_ref[pl.ds(k * TABLE_ROWS, TABLE_ROWS), :] = res
        idx = (t_ref[...] * float(TIME_RES)).astype(jnp.int32)
        r = lax.broadcasted_iota(jnp.int32, idx.shape, 0)
        rep = (r // n_chunks) % N_REP
        idx_ref[...] = idx + rep * TABLE_ROWS

    return _prep_body


def _prep(pe, w1, b1, w2, b2, t_2d, n_chunks):
    n_rows = t_2d.shape[0]
    return pl.pallas_call(
        _make_prep(n_chunks),
        out_shape=(
            jax.ShapeDtypeStruct((N_REP * TABLE_ROWS, LATENT_DIM),
                                 jnp.float32),
            jax.ShapeDtypeStruct((n_rows, IDX_CHUNK), jnp.int32),
        ),
        grid=(1,),
        in_specs=[
            pl.BlockSpec((TABLE_ROWS, 1, LATENT_DIM), lambda i: (0, 0, 0)),
            pl.BlockSpec((LATENT_DIM, LATENT_DIM), lambda i: (0, 0)),
            pl.BlockSpec((1, LATENT_DIM), lambda i: (0, 0)),
            pl.BlockSpec((LATENT_DIM, LATENT_DIM), lambda i: (0, 0)),
            pl.BlockSpec((1, LATENT_DIM), lambda i: (0, 0)),
            pl.BlockSpec((n_rows, IDX_CHUNK), lambda i: (0, 0)),
        ],
        out_specs=(
            pl.BlockSpec((N_REP * TABLE_ROWS, LATENT_DIM), lambda i: (0, 0)),
            pl.BlockSpec((n_rows, IDX_CHUNK), lambda i: (0, 0)),
        ),
    )(pe, w1, b1, w2, b2, t_2d)


@functools.cache
def _gather_fn(batch, n_workers):
    rows_per_w = batch // n_workers
    n_chunks = rows_per_w // IDX_CHUNK
    mesh = plsc.VectorSubcoreMesh(core_axis_name="c", subcore_axis_name="s")

    @functools.partial(
        pl.kernel,
        out_type=jax.ShapeDtypeStruct((batch, LATENT_DIM), jnp.float32),
        mesh=mesh,
        scratch_types=[
            pltpu.VMEM((n_chunks, IDX_CHUNK), jnp.int32),
            pltpu.VMEM((rows_per_w, LATENT_DIM), jnp.float32),
            pltpu.SemaphoreType.DMA,
        ],
    )
    def gather(idx_hbm, table_hbm, out_hbm, idx_v, rows_v, sem):
        wid = lax.axis_index("s") * 2 + lax.axis_index("c")
        base = wid * rows_per_w
        pltpu.sync_copy(idx_hbm.at[pl.ds(wid * n_chunks, n_chunks)], idx_v)
        copies = []
        for c in range(n_chunks):
            copies.append(
                pltpu.async_copy(
                    table_hbm.at[idx_v.at[c]],
                    rows_v.at[pl.ds(c * IDX_CHUNK, IDX_CHUNK)],
                    sem,
                )
            )
        for cp in copies:
            cp.wait()
        pltpu.sync_copy(rows_v, out_hbm.at[pl.ds(base, rows_per_w)])

    return gather


def kernel(timesteps, pe, W1, b1, W2, b2):
    batch = timesteps.shape[0]
    t_2d = timesteps.reshape(batch // IDX_CHUNK, IDX_CHUNK)
    info = plsc.get_sparse_core_info()
    n_workers = info.num_cores * info.num_subcores
    n_chunks = (batch // n_workers) // IDX_CHUNK
    table, idx = _prep(pe, W1, b1.reshape(1, LATENT_DIM),
                       W2, b2.reshape(1, LATENT_DIM), t_2d, n_chunks)
    out = _gather_fn(batch, n_workers)(idx, table)
    return out.reshape(1, batch, LATENT_DIM)


# speedup vs baseline: 1.0118x; 1.0118x over previous
import functools

import jax
import jax.numpy as jnp
from jax import lax
from jax.experimental import pallas as pl
from jax.experimental.pallas import tpu as pltpu
from jax.experimental.pallas import tpu_sc as plsc

LATENT_DIM = 128
TIME_RES = 1000
TABLE_ROWS = 1024
IDX_CHUNK = 128
N_REP = 4


def _make_prep(n_chunks):
    def _prep_body(pe_ref, w1_ref, b1_ref, w2_ref, b2_ref, t_ref,
                   table_ref, idx_ref):
        x = pe_ref[:, 0, :]
        h = jnp.dot(x, w1_ref[...], preferred_element_type=jnp.float32)
        h = h + b1_ref[...]
        h = h * jax.nn.sigmoid(h)
        o = jnp.dot(h, w2_ref[...], preferred_element_type=jnp.float32)
        res = o + b2_ref[...]
        for k in range(N_REP):
            table_ref[pl.ds(k * TABLE_ROWS, TABLE_ROWS), :] = res
        idx = (t_ref[...] * float(TIME_RES)).astype(jnp.int32)
        r = lax.broadcasted_iota(jnp.int32, idx.shape, 0)
        rep = (r // n_chunks) % N_REP
        idx_ref[...] = idx + rep * TABLE_ROWS

    return _prep_body


def _prep(pe, w1, b1, w2, b2, t_2d, n_chunks):
    n_rows = t_2d.shape[0]
    return pl.pallas_call(
        _make_prep(n_chunks),
        out_shape=(
            jax.ShapeDtypeStruct((N_REP * TABLE_ROWS, LATENT_DIM),
                                 jnp.float32),
            jax.ShapeDtypeStruct((n_rows, IDX_CHUNK), jnp.int32),
        ),
        grid=(1,),
        in_specs=[
            pl.BlockSpec((TABLE_ROWS, 1, LATENT_DIM), lambda i: (0, 0, 0)),
            pl.BlockSpec((LATENT_DIM, LATENT_DIM), lambda i: (0, 0)),
            pl.BlockSpec((1, LATENT_DIM), lambda i: (0, 0)),
            pl.BlockSpec((LATENT_DIM, LATENT_DIM), lambda i: (0, 0)),
            pl.BlockSpec((1, LATENT_DIM), lambda i: (0, 0)),
            pl.BlockSpec((n_rows, IDX_CHUNK), lambda i: (0, 0)),
        ],
        out_specs=(
            pl.BlockSpec((N_REP * TABLE_ROWS, LATENT_DIM), lambda i: (0, 0)),
            pl.BlockSpec((n_rows, IDX_CHUNK), lambda i: (0, 0)),
        ),
    )(pe, w1, b1, w2, b2, t_2d)


@functools.cache
def _gather_fn(batch, n_workers):
    rows_per_w = batch // n_workers
    n_chunks = rows_per_w // IDX_CHUNK
    mesh = plsc.VectorSubcoreMesh(core_axis_name="c", subcore_axis_name="s")

    @functools.partial(
        pl.kernel,
        out_type=jax.ShapeDtypeStruct((batch, LATENT_DIM), jnp.float32),
        mesh=mesh,
        scratch_types=[
            pltpu.VMEM((n_chunks, IDX_CHUNK), jnp.int32),
            pltpu.VMEM((rows_per_w, LATENT_DIM), jnp.float32),
            pltpu.SemaphoreType.DMA,
        ],
    )
    def gather(idx_hbm, table_hbm, out_hbm, idx_v, rows_v, sem):
        wid = lax.axis_index("s") * 2 + lax.axis_index("c")
        base = wid * rows_per_w
        pltpu.sync_copy(idx_hbm.at[pl.ds(wid * n_chunks, n_chunks)], idx_v)
        copies = []
        for c in range(n_chunks):
            copies.append(
                pltpu.async_copy(
                    table_hbm.at[idx_v.at[c]],
                    rows_v.at[pl.ds(c * IDX_CHUNK, IDX_CHUNK)],
                    sem,
                )
            )
        for cp in copies:
            cp.wait()
        pltpu.sync_copy(rows_v, out_hbm.at[pl.ds(base, rows_per_w)])

    return gather


def kernel(timesteps, pe, W1, b1, W2, b2):
    batch = timesteps.shape[0]
    t_2d = timesteps.reshape(batch // IDX_CHUNK, IDX_CHUNK)
    info = plsc.get_sparse_core_info()
    n_workers = info.num_cores * info.num_subcores
    n_chunks = (batch // n_workers) // IDX_CHUNK
    table, idx = _prep(pe, W1, b1.reshape(1, LATENT_DIM),
                       W2, b2.reshape(1, LATENT_DIM), t_2d, n_chunks)
    out = _gather_fn(batch, n_workers)(idx, table)
    return out.reshape(1, batch, LATENT_DIM)
